# Initial kernel scaffold; baseline (speedup 1.0000x reference)
import jax
import jax.numpy as jnp
from jax.experimental import pallas as pl


def kernel(x, weight, lora_A, lora_B):
    raise NotImplementedError("write your pallas kernel here")


# speedup vs baseline: 3.2973x; 3.2973x over previous
import functools
import math

import jax
import jax.numpy as jnp
from jax import lax
from jax.experimental import pallas as pl
from jax.experimental.pallas import tpu as pltpu
from jax.experimental.pallas import tpu_sc as plsc

_D = 64
_R = 8
_SCALING = 1.0 / math.sqrt(_R)
_NC, _NS, _LANES = 2, 16, 16
_NW = _NC * _NS
_G = 128
_CH = 512
_NG = _CH // _G
_DV = _D // _LANES


def _lane(vec, lane):
  return jnp.take_along_axis(vec, jnp.full((_LANES,), lane, jnp.int32), axis=0)


@functools.cache
def _build_call(n_rows, interpret=False):
  rows_w = n_rows // _NW
  n_chunks = rows_w // _CH
  mesh = plsc.VectorSubcoreMesh(
      core_axis_name="c", subcore_axis_name="s",
      num_cores=_NC, num_subcores=_NS)

  @functools.partial(
      pl.kernel,
      out_type=jax.ShapeDtypeStruct((n_rows, _D), jnp.float32),
      mesh=mesh,
      scratch_types=[
          [pltpu.VMEM((_G,), jnp.int32) for _ in range(_NG)],
          pltpu.VMEM((_CH, _D), jnp.float32),
          pltpu.VMEM((_CH, _R), jnp.float32),
          pltpu.VMEM((_R * _D,), jnp.float32),
          pltpu.SemaphoreType.DMA,
          pltpu.SemaphoreType.DMA,
      ],
      compiler_params=pltpu.CompilerParams(
          use_tc_tiling_on_sc=False, needs_layout_passes=False),
      interpret=interpret,
  )
  def call(idx_hbm, w_hbm, a_hbm, b_hbm, out_hbm,
           idx_vs, w_v, a_v, b_v, sem_w, sem_a):
    wid = lax.axis_index("s") * _NC + lax.axis_index("c")
    base = wid * rows_w

    pltpu.sync_copy(b_hbm, b_v)
    cvecs = [b_v[pl.ds(r * _D + dv * _LANES, _LANES)] * _SCALING
             for r in range(_R) for dv in range(_DV)]

    iota = lax.iota(jnp.int32, _LANES)
    col_sel = jnp.bitwise_and(iota, _R - 1)
    row_par = lax.shift_right_logical(iota, 3)

    def do_chunk(ci, carry):
      cbase = base + ci * _CH
      for j in range(_NG):
        pltpu.sync_copy(idx_hbm.at[cbase // _G + j], idx_vs[j])
      descs = []
      for j in range(_NG):
        descs.append(pltpu.async_copy(
            w_hbm.at[idx_vs[j]], w_v.at[pl.ds(j * _G, _G)], sem_w))
        descs.append(pltpu.async_copy(
            a_hbm.at[idx_vs[j]], a_v.at[pl.ds(j * _G, _G)], sem_a))
      for d in descs:
        d.wait()

      def do_pair(p, c2):
        av = plsc.load_gather(a_v, [2 * p + row_par, col_sel])
        for half in range(2):
          row = 2 * p + half
          acc = [w_v[row, pl.ds(dv * _LANES, _LANES)] for dv in range(_DV)]
          for r in range(_R):
            b = _lane(av, half * _R + r)
            for dv in range(_DV):
              acc[dv] = acc[dv] + b * cvecs[r * _DV + dv]
          for dv in range(_DV):
            w_v[row, pl.ds(dv * _LANES, _LANES)] = acc[dv]
        return c2

      lax.fori_loop(0, _CH // 2, do_pair, 0)
      pltpu.sync_copy(w_v, out_hbm.at[pl.ds(cbase, _CH)])
      return carry

    lax.fori_loop(0, n_chunks, do_chunk, 0)

  return call


def kernel(x, weight, lora_A, lora_B):
  b, l = x.shape
  n = b * l
  idx = x.reshape(n // _G, _G).astype(jnp.int32)
  b_flat = lora_B.reshape(_R * _D)
  out = _build_call(n)(idx, weight, lora_A, b_flat)
  return out.reshape(b, l, _D)


# speedup vs baseline: 3.4717x; 1.0529x over previous
import functools
import math

import jax
import jax.numpy as jnp
from jax import lax
from jax.experimental import pallas as pl
from jax.experimental.pallas import tpu as pltpu
from jax.experimental.pallas import tpu_sc as plsc

_D = 64
_R = 8
_SCALING = 1.0 / math.sqrt(_R)
_NC, _NS, _LANES = 2, 16, 16
_NW = _NC * _NS
_G = 128
_CH = 512
_NG = _CH // _G
_DV = _D // _LANES


def _lane(vec, lane):
  return jnp.take_along_axis(vec, jnp.full((_LANES,), lane, jnp.int32), axis=0)


@functools.cache
def _build_call(n_rows):
  rows_w = n_rows // _NW
  n_chunks = rows_w // _CH
  mesh = plsc.VectorSubcoreMesh(
      core_axis_name="c", subcore_axis_name="s",
      num_cores=_NC, num_subcores=_NS)

  @functools.partial(
      pl.kernel,
      out_type=jax.ShapeDtypeStruct((n_rows, _D), jnp.float32),
      mesh=mesh,
      scratch_types=[
          [pltpu.VMEM((_NG, _G), jnp.int32) for _ in range(3)],
          [pltpu.VMEM((_CH, _D), jnp.float32) for _ in range(2)],
          [pltpu.VMEM((_CH, _R), jnp.float32) for _ in range(2)],
          pltpu.VMEM((_R * _D,), jnp.float32),
          [pltpu.SemaphoreType.DMA for _ in range(3)],
          [pltpu.SemaphoreType.DMA for _ in range(2)],
          [pltpu.SemaphoreType.DMA for _ in range(2)],
      ],
      compiler_params=pltpu.CompilerParams(
          use_tc_tiling_on_sc=False, needs_layout_passes=False),
  )
  def call(idx_hbm, w_hbm, a_hbm, b_hbm, out_hbm,
           idx_bufs, w_bufs, a_bufs, b_v, sem_i, sem_g, sem_o):
    wid = lax.axis_index("s") * _NC + lax.axis_index("c")
    base = wid * rows_w

    pltpu.sync_copy(b_hbm, b_v)
    cvecs = [b_v[pl.ds(r * _D + dv * _LANES, _LANES)] * _SCALING
             for r in range(_R) for dv in range(_DV)]

    iota = lax.iota(jnp.int32, _LANES)
    col_sel = jnp.bitwise_and(iota, _R - 1)
    row_par = lax.shift_right_logical(iota, 3)

    def fire_idx(c):
      gb = (base + c * _CH) // _G
      return pltpu.async_copy(
          idx_hbm.at[pl.ds(gb, _NG)], idx_bufs[c % 3], sem_i[c % 3])

    def fire_gathers(c):
      b = c % 2
      descs = []
      for j in range(_NG):
        descs.append(pltpu.async_copy(
            w_hbm.at[idx_bufs[c % 3].at[j]],
            w_bufs[b].at[pl.ds(j * _G, _G)], sem_g[b]))
        descs.append(pltpu.async_copy(
            a_hbm.at[idx_bufs[c % 3].at[j]],
            a_bufs[b].at[pl.ds(j * _G, _G)], sem_g[b]))
      return descs

    def fire_out(c):
      b = c % 2
      cbase = base + c * _CH
      return pltpu.async_copy(
          w_bufs[b], out_hbm.at[pl.ds(cbase, _CH)], sem_o[b])

    def compute(c):
      b = c % 2
      w_v, a_v = w_bufs[b], a_bufs[b]

      def do_pair(p, c2):
        av = plsc.load_gather(a_v, [2 * p + row_par, col_sel])
        for half in range(2):
          row = 2 * p + half
          acc = [w_v[row, pl.ds(dv * _LANES, _LANES)] for dv in range(_DV)]
          for r in range(_R):
            bb = _lane(av, half * _R + r)
            for dv in range(_DV):
              acc[dv] = acc[dv] + bb * cvecs[r * _DV + dv]
          for dv in range(_DV):
            w_v[row, pl.ds(dv * _LANES, _LANES)] = acc[dv]
        return c2

      lax.fori_loop(0, _CH // 2, do_pair, 0)

    idx_d = {0: fire_idx(0)}
    if n_chunks > 1:
      idx_d[1] = fire_idx(1)
    idx_d[0].wait()
    g_d = {0: fire_gathers(0)}
    o_d = {}
    for c in range(n_chunks):
      if c + 1 < n_chunks:
        idx_d[c + 1].wait()
        if c - 1 >= 0:
          o_d[c - 1].wait()
        g_d[c + 1] = fire_gathers(c + 1)
      if c + 2 < n_chunks:
        idx_d[c + 2] = fire_idx(c + 2)
      for d in g_d.pop(c):
        d.wait()
      compute(c)
      o_d[c] = fire_out(c)
    o_d[n_chunks - 1].wait()
    if n_chunks > 1:
      o_d[n_chunks - 2].wait()

  return call


def kernel(x, weight, lora_A, lora_B):
  b, l = x.shape
  n = b * l
  idx = x.reshape(n // _G, _G).astype(jnp.int32)
  b_flat = lora_B.reshape(_R * _D)
  out = _build_call(n)(idx, weight, lora_A, b_flat)
  return out.reshape(b, l, _D)


# speedup vs baseline: 3.9752x; 1.1450x over previous
import functools
import math

import jax
import jax.numpy as jnp
from jax import lax
from jax.experimental import pallas as pl
from jax.experimental.pallas import tpu as pltpu
from jax.experimental.pallas import tpu_sc as plsc

_D = 64
_DP = 128
_R = 8
_SCALING = 1.0 / math.sqrt(_R)
_NC, _NS, _LANES = 2, 16, 16
_NW = _NC * _NS
_G = 128
_CH = 256
_NG = _CH // _G
_DV = _D // _LANES
_CB = 3584


def _lane(vec, lane):
  return jnp.take_along_axis(vec, jnp.full((_LANES,), lane, jnp.int32), axis=0)


@functools.cache
def _build_transpose(v):
  grid = (v + _CB - 1) // _CB
  v2 = grid * _CB

  def body(in_ref, out_ref):
    t = in_ref[...].T
    out_ref[:, 0:_D] = t
    out_ref[:, _D:_DP] = jnp.zeros((_CB, _DP - _D), jnp.float32)

  return pl.pallas_call(
      body,
      grid=(grid,),
      in_specs=[pl.BlockSpec((_D, _CB), lambda k: (0, k))],
      out_specs=pl.BlockSpec((_CB, _DP), lambda k: (k, 0)),
      out_shape=jax.ShapeDtypeStruct((v2, _DP), jnp.float32),
  )


@functools.cache
def _build_call(n_rows):
  rows_w = n_rows // _NW
  n_chunks = rows_w // _CH
  mesh = plsc.VectorSubcoreMesh(
      core_axis_name="c", subcore_axis_name="s",
      num_cores=_NC, num_subcores=_NS)

  @functools.partial(
      pl.kernel,
      out_type=jax.ShapeDtypeStruct((n_rows, _D), jnp.float32),
      mesh=mesh,
      scratch_types=[
          [pltpu.VMEM((_NG, _G), jnp.int32) for _ in range(3)],
          [pltpu.VMEM((_CH, _DP), jnp.float32) for _ in range(2)],
          [pltpu.VMEM((_CH, _D), jnp.float32) for _ in range(2)],
          [pltpu.VMEM((_CH, _R), jnp.float32) for _ in range(2)],
          pltpu.VMEM((_R * _D,), jnp.float32),
          [pltpu.SemaphoreType.DMA for _ in range(3)],
          [pltpu.SemaphoreType.DMA for _ in range(2)],
          [pltpu.SemaphoreType.DMA for _ in range(2)],
      ],
      compiler_params=pltpu.CompilerParams(
          use_tc_tiling_on_sc=False, needs_layout_passes=False),
  )
  def call(idx_hbm, w_hbm, a_hbm, b_hbm, out_hbm,
           idx_bufs, w_bufs, o_bufs, a_bufs, b_v, sem_i, sem_g, sem_o):
    wid = lax.axis_index("s") * _NC + lax.axis_index("c")
    base = wid * rows_w

    pltpu.sync_copy(b_hbm, b_v)
    cvecs = [b_v[pl.ds(r * _D + dv * _LANES, _LANES)] * _SCALING
             for r in range(_R) for dv in range(_DV)]

    iota = lax.iota(jnp.int32, _LANES)
    col_sel = jnp.bitwise_and(iota, _R - 1)
    row_par = lax.shift_right_logical(iota, 3)

    def fire_idx(c):
      gb = (base + c * _CH) // _G
      return pltpu.async_copy(
          idx_hbm.at[pl.ds(gb, _NG)], idx_bufs[c % 3], sem_i[c % 3])

    def fire_gathers(c):
      b = c % 2
      descs = []
      for j in range(_NG):
        descs.append(pltpu.async_copy(
            w_hbm.at[idx_bufs[c % 3].at[j]],
            w_bufs[b].at[pl.ds(j * _G, _G)], sem_g[b]))
        descs.append(pltpu.async_copy(
            a_hbm.at[idx_bufs[c % 3].at[j]],
            a_bufs[b].at[pl.ds(j * _G, _G)], sem_g[b]))
      return descs

    def fire_out(c):
      b = c % 2
      cbase = base + c * _CH
      return pltpu.async_copy(
          o_bufs[b], out_hbm.at[pl.ds(cbase, _CH)], sem_o[b])

    def compute(c):
      b = c % 2
      w_v, o_v, a_v = w_bufs[b], o_bufs[b], a_bufs[b]

      def do_pair(p, c2):
        av = plsc.load_gather(a_v, [2 * p + row_par, col_sel])
        for half in range(2):
          row = 2 * p + half
          acc = [w_v[row, pl.ds(dv * _LANES, _LANES)] for dv in range(_DV)]
          for r in range(_R):
            bb = _lane(av, half * _R + r)
            for dv in range(_DV):
              acc[dv] = acc[dv] + bb * cvecs[r * _DV + dv]
          for dv in range(_DV):
            o_v[row, pl.ds(dv * _LANES, _LANES)] = acc[dv]
        return c2

      lax.fori_loop(0, _CH // 2, do_pair, 0)

    idx_d = {0: fire_idx(0)}
    if n_chunks > 1:
      idx_d[1] = fire_idx(1)
    idx_d[0].wait()
    g_d = {0: fire_gathers(0)}
    o_d = {}
    for c in range(n_chunks):
      if c + 1 < n_chunks:
        idx_d[c + 1].wait()
        if c - 1 >= 0:
          o_d[c - 1].wait()
        g_d[c + 1] = fire_gathers(c + 1)
      if c + 2 < n_chunks:
        idx_d[c + 2] = fire_idx(c + 2)
      for d in g_d.pop(c):
        d.wait()
      compute(c)
      o_d[c] = fire_out(c)
    o_d[n_chunks - 1].wait()
    if n_chunks > 1:
      o_d[n_chunks - 2].wait()

  return call


def kernel(x, weight, lora_A, lora_B):
  b, l = x.shape
  n = b * l
  v, _ = weight.shape
  idx = x.reshape(n // _G, _G).astype(jnp.int32)
  b_flat = lora_B.reshape(_R * _D)
  w_pad = _build_transpose(v)(weight.T)
  out = _build_call(n)(idx, w_pad, lora_A, b_flat)
  return out.reshape(b, l, _D)


# speedup vs baseline: 6.1791x; 1.5544x over previous
import functools
import math

import jax
import jax.numpy as jnp
from jax import lax
from jax.experimental import pallas as pl
from jax.experimental.pallas import tpu as pltpu
from jax.experimental.pallas import tpu_sc as plsc

_D = 64
_DP = 128
_R = 8
_SCALING = 1.0 / math.sqrt(_R)
_NC, _NS, _LANES = 2, 16, 16
_NW = _NC * _NS
_G = 128
_CH = 256
_NG = _CH // _G
_CB = 3584


@functools.cache
def _build_repack(v):
  grid = (v + _CB - 1) // _CB
  v2 = grid * _CB

  def body(w_ref, a_ref, b_ref, out_ref):
    wt = w_ref[...].T
    at = a_ref[...].T
    bs = b_ref[...] * _SCALING
    out_ref[:, 0:_D] = wt + jnp.dot(
        at, bs, preferred_element_type=jnp.float32)
    out_ref[:, _D:_DP] = jnp.zeros((_CB, _DP - _D), jnp.float32)

  return pl.pallas_call(
      body,
      grid=(grid,),
      in_specs=[
          pl.BlockSpec((_D, _CB), lambda k: (0, k)),
          pl.BlockSpec((_R, _CB), lambda k: (0, k)),
          pl.BlockSpec((_R, _D), lambda k: (0, 0)),
      ],
      out_specs=pl.BlockSpec((_CB, _DP), lambda k: (k, 0)),
      out_shape=jax.ShapeDtypeStruct((v2, _DP), jnp.float32),
  )


@functools.cache
def _build_gather(n_rows):
  rows_w = n_rows // _NW
  n_chunks = rows_w // _CH
  mesh = plsc.VectorSubcoreMesh(
      core_axis_name="c", subcore_axis_name="s",
      num_cores=_NC, num_subcores=_NS)

  @functools.partial(
      pl.kernel,
      out_type=jax.ShapeDtypeStruct((n_rows, _D), jnp.float32),
      mesh=mesh,
      scratch_types=[
          [pltpu.VMEM((_NG, _G), jnp.int32) for _ in range(3)],
          [pltpu.VMEM((_CH, _DP), jnp.float32) for _ in range(2)],
          [pltpu.SemaphoreType.DMA for _ in range(3)],
          [pltpu.SemaphoreType.DMA for _ in range(2)],
          [pltpu.SemaphoreType.DMA for _ in range(2)],
      ],
      compiler_params=pltpu.CompilerParams(
          use_tc_tiling_on_sc=False, needs_layout_passes=False),
  )
  def call(idx_hbm, w_hbm, out_hbm, idx_bufs, w_bufs, sem_i, sem_g, sem_o):
    wid = lax.axis_index("s") * _NC + lax.axis_index("c")
    base = wid * rows_w

    def fire_idx(c):
      gb = (base + c * _CH) // _G
      return pltpu.async_copy(
          idx_hbm.at[pl.ds(gb, _NG)], idx_bufs[c % 3], sem_i[c % 3])

    def fire_gathers(c):
      b = c % 2
      return [pltpu.async_copy(
          w_hbm.at[idx_bufs[c % 3].at[j]],
          w_bufs[b].at[pl.ds(j * _G, _G)], sem_g[b]) for j in range(_NG)]

    def fire_out(c):
      b = c % 2
      cbase = base + c * _CH
      return pltpu.async_copy(
          w_bufs[b].at[:, pl.ds(0, _D)],
          out_hbm.at[pl.ds(cbase, _CH)], sem_o[b])

    idx_d = {0: fire_idx(0)}
    if n_chunks > 1:
      idx_d[1] = fire_idx(1)
    idx_d[0].wait()
    g_d = {0: fire_gathers(0)}
    o_d = {}
    for c in range(n_chunks):
      if c + 1 < n_chunks:
        idx_d[c + 1].wait()
        if c - 1 >= 0:
          o_d[c - 1].wait()
        g_d[c + 1] = fire_gathers(c + 1)
      if c + 2 < n_chunks:
        idx_d[c + 2] = fire_idx(c + 2)
      for d in g_d.pop(c):
        d.wait()
      o_d[c] = fire_out(c)
    o_d[n_chunks - 1].wait()
    if n_chunks > 1:
      o_d[n_chunks - 2].wait()

  return call


def kernel(x, weight, lora_A, lora_B):
  b, l = x.shape
  n = b * l
  v, _ = weight.shape
  idx = x.reshape(n // _G, _G).astype(jnp.int32)
  w_pad = _build_repack(v)(weight.T, lora_A.T, lora_B)
  out = _build_gather(n)(idx, w_pad)
  return out.reshape(b, l, _D)
